# baseline (speedup 1.0000x reference)
import functools
import math

import jax
import jax.numpy as jnp
from jax import lax
from jax.experimental import pallas as pl
from jax.experimental.pallas import tpu as pltpu
from jax.experimental.pallas import tpu_sc as plsc

R_MAX = 5.0
NUM_BESSEL = 8
N_NODES = 10000
N_EDGES = 320000
D = 128
DH = 64
N_GRAPHS = 64
AVG_NEIGH = 12.0

BE = 4096
E_PAD = 331776
BN = 2000
GN = N_NODES // BN

EB = 96
NCH = E_PAD // (16 * EB)
N_PAD = 10240
ROWS_PER_TILE = N_PAD // 16


def _edge_w_body(r2_ref, w1a_ref, w2a_ref, w1b_ref, w2b_ref, ew0_ref, ew1_ref):
    r2 = r2_ref[...]
    r = jnp.sqrt(r2)
    x = r * (1.0 / R_MAX)
    x5 = x * x * x * x * x
    env = 1.0 - 21.0 * x5 + 35.0 * x5 * x - 15.0 * x5 * x * x
    env = jnp.where(x < 1.0, env, 0.0)
    scale = math.sqrt(2.0 / R_MAX) * env / r
    n_i = jax.lax.broadcasted_iota(jnp.int32, (1, NUM_BESSEL), 1) + 1
    n = n_i.astype(jnp.float32) * (math.pi / R_MAX)
    rbf = jnp.sin(r[:, None] * n) * scale[:, None]
    t0 = jnp.maximum(jnp.dot(rbf, w1a_ref[...], preferred_element_type=jnp.float32), 0.0)
    ew0 = jnp.dot(t0, w2a_ref[...], preferred_element_type=jnp.float32)
    ew0_ref[0] = ew0[:, :DH]
    ew0_ref[1] = ew0[:, DH:]
    t1 = jnp.maximum(jnp.dot(rbf, w1b_ref[...], preferred_element_type=jnp.float32), 0.0)
    ew1 = jnp.dot(t1, w2b_ref[...], preferred_element_type=jnp.float32)
    ew1_ref[0] = ew1[:, :DH]
    ew1_ref[1] = ew1[:, DH:]


def _edge_weights(r2, W1_0, W2_0, W1_1, W2_1):
    grid = E_PAD // BE
    return pl.pallas_call(
        _edge_w_body,
        grid=(grid,),
        in_specs=[
            pl.BlockSpec((BE,), lambda i: (i,)),
            pl.BlockSpec((NUM_BESSEL, 64), lambda i: (0, 0)),
            pl.BlockSpec((64, D), lambda i: (0, 0)),
            pl.BlockSpec((NUM_BESSEL, 64), lambda i: (0, 0)),
            pl.BlockSpec((64, D), lambda i: (0, 0)),
        ],
        out_specs=[
            pl.BlockSpec((2, BE, DH), lambda i: (0, i, 0)),
            pl.BlockSpec((2, BE, DH), lambda i: (0, i, 0)),
        ],
        out_shape=[
            jax.ShapeDtypeStruct((2, E_PAD, DH), jnp.float32),
            jax.ShapeDtypeStruct((2, E_PAD, DH), jnp.float32),
        ],
    )(r2, W1_0, W2_0, W1_1, W2_1)


def _update_body(hs_in_ref, aggs_ref, wlin_ref, hs_ref):
    h = jnp.concatenate([hs_in_ref[0], hs_in_ref[1]], axis=-1)
    agg = jnp.concatenate([aggs_ref[0], aggs_ref[1]], axis=-1) * (1.0 / AVG_NEIGH)
    h1 = h + jnp.dot(agg, wlin_ref[...], preferred_element_type=jnp.float32)
    hs_ref[0] = h1[:, :DH]
    hs_ref[1] = h1[:, DH:]


def _update(hs, aggs, Wlin):
    return pl.pallas_call(
        _update_body,
        grid=(GN,),
        in_specs=[
            pl.BlockSpec((2, BN, DH), lambda i: (0, i, 0)),
            pl.BlockSpec((2, BN, DH), lambda i: (0, i, 0)),
            pl.BlockSpec((D, D), lambda i: (0, 0)),
        ],
        out_specs=pl.BlockSpec((2, BN, DH), lambda i: (0, i, 0)),
        out_shape=jax.ShapeDtypeStruct((2, N_PAD, DH), jnp.float32),
    )(hs, aggs, Wlin)


def _readout_body(hs_in_ref, aggs_ref, wlin_ref, wr1_ref, wr2_ref,
                  batch_ref, out_ref):
    i = pl.program_id(0)
    h = jnp.concatenate([hs_in_ref[0], hs_in_ref[1]], axis=-1)
    agg = jnp.concatenate([aggs_ref[0], aggs_ref[1]], axis=-1) * (1.0 / AVG_NEIGH)
    h2 = h + jnp.dot(agg, wlin_ref[...], preferred_element_type=jnp.float32)
    t = jnp.dot(h2, wr1_ref[...], preferred_element_type=jnp.float32)
    t = t * jax.nn.sigmoid(t)
    e = jnp.dot(t, wr2_ref[...], preferred_element_type=jnp.float32)
    b = batch_ref[0, 0, :]
    onehot = (b[:, None] == jax.lax.broadcasted_iota(jnp.int32, (BN, N_GRAPHS), 1))
    contrib = jnp.dot(e[:, 0][None, :], onehot.astype(jnp.float32),
                      preferred_element_type=jnp.float32)

    @pl.when(i == 0)
    def _init():
        out_ref[...] = jnp.zeros_like(out_ref)

    out_ref[...] += contrib


def _readout(hs, aggs, Wlin, Wr1, Wr2, batch):
    wr2p = jnp.pad(Wr2, ((0, 0), (0, 127)))
    batch3 = batch.reshape(GN, 1, BN)
    out = pl.pallas_call(
        _readout_body,
        grid=(GN,),
        in_specs=[
            pl.BlockSpec((2, BN, DH), lambda i: (0, i, 0)),
            pl.BlockSpec((2, BN, DH), lambda i: (0, i, 0)),
            pl.BlockSpec((D, D), lambda i: (0, 0)),
            pl.BlockSpec((D, 16), lambda i: (0, 0)),
            pl.BlockSpec((16, D), lambda i: (0, 0)),
            pl.BlockSpec((1, 1, BN), lambda i: (i, 0, 0)),
        ],
        out_specs=pl.BlockSpec((1, N_GRAPHS), lambda i: (0, 0)),
        out_shape=jax.ShapeDtypeStruct((1, N_GRAPHS), jnp.float32),
    )(hs, aggs, Wlin, Wr1, wr2p, batch3)
    return out[0]


EPW = E_PAD // 32
NPW = N_PAD // 32


def _geom_body(post_hbm, ei_hbm, an_hbm, embs_hbm, r2_hbm, h0s_hbm,
               px, py, pz, sidx, didx, r2b, anb, hrows, sem):
    cid = lax.axis_index("c")
    sid = lax.axis_index("s")
    w = cid * 16 + sid

    for hf in (0, 1):
        pltpu.sync_copy(an_hbm.at[hf, w], anb)
        for c in range(4):
            pltpu.async_copy(embs_hbm.at[anb.at[c]], hrows, sem).wait()
            pltpu.sync_copy(hrows, h0s_hbm.at[hf, pl.ds(w * NPW + c * 80, 80)])

    pltpu.sync_copy(post_hbm.at[0], px)
    pltpu.sync_copy(post_hbm.at[1], py)
    pltpu.sync_copy(post_hbm.at[2], pz)
    pltpu.sync_copy(ei_hbm.at[0, w], sidx)
    pltpu.sync_copy(ei_hbm.at[1, w], didx)

    def _g(i, carry):
        sl = pl.ds(i * 16, 16)
        s16 = sidx[sl]
        d16 = didx[sl]
        dx = plsc.load_gather(px, [d16]) - plsc.load_gather(px, [s16])
        dy = plsc.load_gather(py, [d16]) - plsc.load_gather(py, [s16])
        dz = plsc.load_gather(pz, [d16]) - plsc.load_gather(pz, [s16])
        r2b[sl] = dx * dx + dy * dy + dz * dz + 1e-12
        return carry
    lax.fori_loop(0, EPW // 16, _g, 0)
    pltpu.sync_copy(r2b, r2_hbm.at[pl.ds(w * EPW, EPW)])


@functools.partial(
    pl.kernel,
    out_type=[
        jax.ShapeDtypeStruct((E_PAD,), jnp.float32),
        jax.ShapeDtypeStruct((2, N_PAD, DH), jnp.float32),
    ],
    mesh=plsc.VectorSubcoreMesh(core_axis_name="c", subcore_axis_name="s"),
    compiler_params=pltpu.CompilerParams(use_tc_tiling_on_sc=False,
                                         needs_layout_passes=False),
    scratch_types=[
        pltpu.VMEM((N_NODES,), jnp.float32),
        pltpu.VMEM((N_NODES,), jnp.float32),
        pltpu.VMEM((N_NODES,), jnp.float32),
        pltpu.VMEM((EPW,), jnp.int32),
        pltpu.VMEM((EPW,), jnp.int32),
        pltpu.VMEM((EPW,), jnp.float32),
        pltpu.VMEM((4, 80), jnp.int32),
        pltpu.VMEM((80, DH), jnp.float32),
        pltpu.SemaphoreType.DMA,
    ],
)
def _geometry(post_hbm, ei_hbm, an_hbm, embs_hbm, r2_hbm, h0s_hbm,
              px, py, pz, sidx, didx, r2b, anb, hrows, sem):
    _geom_body(post_hbm, ei_hbm, an_hbm, embs_hbm, r2_hbm, h0s_hbm,
               px, py, pz, sidx, didx, r2b, anb, hrows, sem)


def _mp_body(h_hbm, ew_hbm, srcr_hbm, dstr_hbm, out_hbm,
             idx_s, idx_d, rows0, rows1, rows2, ewv0, ewv1, ewv2,
             agg_sh,
             sg0, sg1, sg2, sw0, sw1, sw2, ss0, ss1, ss2):
    cid = lax.axis_index("c")
    sid = lax.axis_index("s")
    rows = (rows0, rows1, rows2)
    ews = (ewv0, ewv1, ewv2)
    sgs = (sg0, sg1, sg2)
    sws = (sw0, sw1, sw2)
    sss = (ss0, ss1, ss2)

    def _zero_z(i, carry):
        for k in range(4):
            rows0[i, pl.ds(k * 16, 16)] = jnp.zeros((16,), jnp.float32)
        return carry
    lax.fori_loop(0, 64, _zero_z, 0)
    for j in range(10):
        pltpu.sync_copy(rows0.at[pl.ds(0, 64)],
                        agg_sh.at[pl.ds(sid * ROWS_PER_TILE + j * 64, 64)])
    plsc.subcore_barrier()

    pltpu.sync_copy(srcr_hbm.at[cid, sid], idx_s)
    pltpu.sync_copy(dstr_hbm.at[sid], idx_d)

    def _ew_src(c):
        return ew_hbm.at[cid, pl.ds(sid * (NCH * EB) + c * EB, EB)]

    def _issue(b, c):
        pltpu.async_copy(h_hbm.at[idx_s.at[c]], rows[b], sgs[b])
        pltpu.async_copy(_ew_src(c), ews[b], sws[b])

    for b in (0, 1):
        _issue(b, b)

    def _one(b, bp, c):
        pltpu.make_async_copy(h_hbm.at[idx_s.at[c]], rows[b], sgs[b]).wait()
        pltpu.make_async_copy(_ew_src(c), ews[b], sws[b]).wait()

        def _mul(i, carry2):
            for u in range(8):
                for k in range(4):
                    sl = pl.ds(k * 16, 16)
                    rows[b][i * 8 + u, sl] = (rows[b][i * 8 + u, sl]
                                              * ews[b][i * 8 + u, sl])
            return carry2
        lax.fori_loop(0, EB // 8, _mul, 0)
        pltpu.async_copy(rows[b], agg_sh.at[idx_d.at[c]], sss[b], add=True)

        @pl.when(c >= 1)
        def _drain_prev():
            pltpu.make_async_copy(rows[bp], agg_sh.at[idx_d.at[c]],
                                  sss[bp]).wait()

        @pl.when(c + 2 < NCH)
        def _next():
            _issue(bp, c + 2)

    def _six(g, carry):
        base = 6 * g
        for u in range(6):
            _one(u % 3, (u + 2) % 3, base + u)
        return carry
    lax.fori_loop(0, NCH // 6, _six, 0)
    b_last = (NCH - 1) % 3
    pltpu.make_async_copy(rows[b_last], agg_sh.at[idx_d.at[NCH - 1]],
                          sss[b_last]).wait()

    plsc.subcore_barrier()
    pltpu.sync_copy(agg_sh.at[pl.ds(sid * ROWS_PER_TILE, ROWS_PER_TILE)],
                    out_hbm.at[cid, pl.ds(sid * ROWS_PER_TILE, ROWS_PER_TILE)])


@functools.partial(
    pl.kernel,
    out_type=jax.ShapeDtypeStruct((2, N_PAD, DH), jnp.float32),
    mesh=plsc.VectorSubcoreMesh(core_axis_name="c", subcore_axis_name="s"),
    compiler_params=pltpu.CompilerParams(use_tc_tiling_on_sc=False),
    scratch_types=[
        pltpu.VMEM((NCH, EB), jnp.int32),
        pltpu.VMEM((NCH, EB), jnp.int32),
        pltpu.VMEM((EB, DH), jnp.float32),
        pltpu.VMEM((EB, DH), jnp.float32),
        pltpu.VMEM((EB, DH), jnp.float32),
        pltpu.VMEM((EB, DH), jnp.float32),
        pltpu.VMEM((EB, DH), jnp.float32),
        pltpu.VMEM((EB, DH), jnp.float32),
        pltpu.VMEM_SHARED((N_PAD, DH), jnp.float32),
        pltpu.SemaphoreType.DMA,
        pltpu.SemaphoreType.DMA,
        pltpu.SemaphoreType.DMA,
        pltpu.SemaphoreType.DMA,
        pltpu.SemaphoreType.DMA,
        pltpu.SemaphoreType.DMA,
        pltpu.SemaphoreType.DMA,
        pltpu.SemaphoreType.DMA,
        pltpu.SemaphoreType.DMA,
    ],
)
def _message_pass(h_hbm, ew_hbm, srcr_hbm, dstr_hbm, out_hbm,
                  idx_s, idx_d, rows0, rows1, rows2, ewv0, ewv1, ewv2,
                  agg_sh,
                  sg0, sg1, sg2, sw0, sw1, sw2, ss0, ss1, ss2):
    _mp_body(h_hbm, ew_hbm, srcr_hbm, dstr_hbm, out_hbm,
             idx_s, idx_d, rows0, rows1, rows2, ewv0, ewv1, ewv2,
             agg_sh,
             sg0, sg1, sg2, sw0, sw1, sw2, ss0, ss1, ss2)


def kernel(positions, atomic_numbers, edge_index, batch, embed,
           W1_0, W2_0, Wlin_0, W1_1, W2_1, Wlin_1, Wr1, Wr2):
    src = edge_index[0]
    dst = edge_index[1]
    src_p = jnp.concatenate([src, jnp.zeros((E_PAD - N_EDGES,), src.dtype)])
    dst_p = jnp.concatenate([dst, jnp.full((E_PAD - N_EDGES,), N_NODES,
                                           dst.dtype)])
    pos_t = positions.T
    dst_g = jnp.concatenate([dst, jnp.zeros((E_PAD - N_EDGES,), dst.dtype)])
    ei = jnp.stack([src_p, dst_g]).reshape(2, 32, EPW)
    an_p = jnp.pad(atomic_numbers, (0, N_PAD - N_NODES)).reshape(32, 4, 80)
    an2 = jnp.stack([an_p, an_p + 100])
    embs = jnp.concatenate([embed[:, :DH], embed[:, DH:]], axis=0)
    srcs = src_p.reshape(16, NCH, EB)
    srcr = jnp.stack([srcs, srcs + N_PAD])
    dstr = dst_p.reshape(16, NCH, EB)
    r2, h0s = _geometry(pos_t, ei, an2, embs)
    ew0, ew1 = _edge_weights(r2, W1_0, W2_0, W1_1, W2_1)
    agg0 = _message_pass(h0s.reshape(2 * N_PAD, DH), ew0, srcr, dstr)
    h1s = _update(h0s, agg0, Wlin_0)
    agg1 = _message_pass(h1s.reshape(2 * N_PAD, DH), ew1, srcr, dstr)
    return _readout(h1s, agg1, Wlin_1, Wr1, Wr2, batch)


# speedup vs baseline: 1.1060x; 1.1060x over previous
import functools
import math

import jax
import jax.numpy as jnp
from jax import lax
from jax.experimental import pallas as pl
from jax.experimental.pallas import tpu as pltpu
from jax.experimental.pallas import tpu_sc as plsc

R_MAX = 5.0
NUM_BESSEL = 8
N_NODES = 10000
N_EDGES = 320000
D = 128
DH = 64
N_GRAPHS = 64
AVG_NEIGH = 12.0

BE = 4096
E_PAD = 327680
BN = 2000
GN = N_NODES // BN

EB = 128
NCH = E_PAD // (16 * EB)
N_PAD = 10240
ROWS_PER_TILE = N_PAD // 16


def _edge_w_body(r2_ref, w1a_ref, w2a_ref, w1b_ref, w2b_ref, ew0_ref, ew1_ref):
    r2 = r2_ref[...]
    r = jnp.sqrt(r2)
    x = r * (1.0 / R_MAX)
    x5 = x * x * x * x * x
    env = 1.0 - 21.0 * x5 + 35.0 * x5 * x - 15.0 * x5 * x * x
    env = jnp.where(x < 1.0, env, 0.0)
    scale = math.sqrt(2.0 / R_MAX) * env / r
    n_i = jax.lax.broadcasted_iota(jnp.int32, (1, NUM_BESSEL), 1) + 1
    n = n_i.astype(jnp.float32) * (math.pi / R_MAX)
    rbf = jnp.sin(r[:, None] * n) * scale[:, None]
    t0 = jnp.maximum(jnp.dot(rbf, w1a_ref[...], preferred_element_type=jnp.float32), 0.0)
    ew0 = jnp.dot(t0, w2a_ref[...], preferred_element_type=jnp.float32)
    ew0_ref[0] = ew0[:, :DH]
    ew0_ref[1] = ew0[:, DH:]
    t1 = jnp.maximum(jnp.dot(rbf, w1b_ref[...], preferred_element_type=jnp.float32), 0.0)
    ew1 = jnp.dot(t1, w2b_ref[...], preferred_element_type=jnp.float32)
    ew1_ref[0] = ew1[:, :DH]
    ew1_ref[1] = ew1[:, DH:]


def _edge_weights(r2, W1_0, W2_0, W1_1, W2_1):
    grid = E_PAD // BE
    return pl.pallas_call(
        _edge_w_body,
        grid=(grid,),
        in_specs=[
            pl.BlockSpec((BE,), lambda i: (i,)),
            pl.BlockSpec((NUM_BESSEL, 64), lambda i: (0, 0)),
            pl.BlockSpec((64, D), lambda i: (0, 0)),
            pl.BlockSpec((NUM_BESSEL, 64), lambda i: (0, 0)),
            pl.BlockSpec((64, D), lambda i: (0, 0)),
        ],
        out_specs=[
            pl.BlockSpec((2, BE, DH), lambda i: (0, i, 0)),
            pl.BlockSpec((2, BE, DH), lambda i: (0, i, 0)),
        ],
        out_shape=[
            jax.ShapeDtypeStruct((2, E_PAD, DH), jnp.float32),
            jax.ShapeDtypeStruct((2, E_PAD, DH), jnp.float32),
        ],
    )(r2, W1_0, W2_0, W1_1, W2_1)


def _update_body(hs_in_ref, aggs_ref, wlin_ref, hs_ref):
    h = jnp.concatenate([hs_in_ref[0], hs_in_ref[1]], axis=-1)
    agg = jnp.concatenate([aggs_ref[0], aggs_ref[1]], axis=-1) * (1.0 / AVG_NEIGH)
    h1 = h + jnp.dot(agg, wlin_ref[...], preferred_element_type=jnp.float32)
    hs_ref[0] = h1[:, :DH]
    hs_ref[1] = h1[:, DH:]


def _update(hs, aggs, Wlin):
    return pl.pallas_call(
        _update_body,
        grid=(GN,),
        in_specs=[
            pl.BlockSpec((2, BN, DH), lambda i: (0, i, 0)),
            pl.BlockSpec((2, BN, DH), lambda i: (0, i, 0)),
            pl.BlockSpec((D, D), lambda i: (0, 0)),
        ],
        out_specs=pl.BlockSpec((2, BN, DH), lambda i: (0, i, 0)),
        out_shape=jax.ShapeDtypeStruct((2, N_PAD, DH), jnp.float32),
    )(hs, aggs, Wlin)


def _readout_body(hs_in_ref, aggs_ref, wlin_ref, wr1_ref, wr2_ref,
                  batch_ref, out_ref):
    i = pl.program_id(0)
    h = jnp.concatenate([hs_in_ref[0], hs_in_ref[1]], axis=-1)
    agg = jnp.concatenate([aggs_ref[0], aggs_ref[1]], axis=-1) * (1.0 / AVG_NEIGH)
    h2 = h + jnp.dot(agg, wlin_ref[...], preferred_element_type=jnp.float32)
    t = jnp.dot(h2, wr1_ref[...], preferred_element_type=jnp.float32)
    t = t * jax.nn.sigmoid(t)
    e = jnp.dot(t, wr2_ref[...], preferred_element_type=jnp.float32)
    b = batch_ref[0, 0, :]
    onehot = (b[:, None] == jax.lax.broadcasted_iota(jnp.int32, (BN, N_GRAPHS), 1))
    contrib = jnp.dot(e[:, 0][None, :], onehot.astype(jnp.float32),
                      preferred_element_type=jnp.float32)

    @pl.when(i == 0)
    def _init():
        out_ref[...] = jnp.zeros_like(out_ref)

    out_ref[...] += contrib


def _readout(hs, aggs, Wlin, Wr1, Wr2, batch):
    wr2p = jnp.pad(Wr2, ((0, 0), (0, 127)))
    batch3 = batch.reshape(GN, 1, BN)
    out = pl.pallas_call(
        _readout_body,
        grid=(GN,),
        in_specs=[
            pl.BlockSpec((2, BN, DH), lambda i: (0, i, 0)),
            pl.BlockSpec((2, BN, DH), lambda i: (0, i, 0)),
            pl.BlockSpec((D, D), lambda i: (0, 0)),
            pl.BlockSpec((D, 16), lambda i: (0, 0)),
            pl.BlockSpec((16, D), lambda i: (0, 0)),
            pl.BlockSpec((1, 1, BN), lambda i: (i, 0, 0)),
        ],
        out_specs=pl.BlockSpec((1, N_GRAPHS), lambda i: (0, 0)),
        out_shape=jax.ShapeDtypeStruct((1, N_GRAPHS), jnp.float32),
    )(hs, aggs, Wlin, Wr1, wr2p, batch3)
    return out[0]


EPW = E_PAD // 32
NPW = N_PAD // 32


def _geom_body(post_hbm, ei_hbm, an_hbm, embs_hbm, r2_hbm, h0s_hbm,
               px, py, pz, sidx, didx, r2b, anb, hrows, sem):
    cid = lax.axis_index("c")
    sid = lax.axis_index("s")
    w = cid * 16 + sid

    for hf in (0, 1):
        pltpu.sync_copy(an_hbm.at[hf, w], anb)
        for c in range(4):
            pltpu.async_copy(embs_hbm.at[anb.at[c]], hrows, sem).wait()
            pltpu.sync_copy(hrows, h0s_hbm.at[hf, pl.ds(w * NPW + c * 80, 80)])

    pltpu.sync_copy(post_hbm.at[0], px)
    pltpu.sync_copy(post_hbm.at[1], py)
    pltpu.sync_copy(post_hbm.at[2], pz)
    pltpu.sync_copy(ei_hbm.at[0, w], sidx)
    pltpu.sync_copy(ei_hbm.at[1, w], didx)

    def _g(i, carry):
        sl = pl.ds(i * 16, 16)
        s16 = sidx[sl]
        d16 = didx[sl]
        dx = plsc.load_gather(px, [d16]) - plsc.load_gather(px, [s16])
        dy = plsc.load_gather(py, [d16]) - plsc.load_gather(py, [s16])
        dz = plsc.load_gather(pz, [d16]) - plsc.load_gather(pz, [s16])
        r2b[sl] = dx * dx + dy * dy + dz * dz + 1e-12
        return carry
    lax.fori_loop(0, EPW // 16, _g, 0)
    pltpu.sync_copy(r2b, r2_hbm.at[pl.ds(w * EPW, EPW)])


@functools.partial(
    pl.kernel,
    out_type=[
        jax.ShapeDtypeStruct((E_PAD,), jnp.float32),
        jax.ShapeDtypeStruct((2, N_PAD, DH), jnp.float32),
    ],
    mesh=plsc.VectorSubcoreMesh(core_axis_name="c", subcore_axis_name="s"),
    compiler_params=pltpu.CompilerParams(use_tc_tiling_on_sc=False,
                                         needs_layout_passes=False),
    scratch_types=[
        pltpu.VMEM((N_NODES,), jnp.float32),
        pltpu.VMEM((N_NODES,), jnp.float32),
        pltpu.VMEM((N_NODES,), jnp.float32),
        pltpu.VMEM((EPW,), jnp.int32),
        pltpu.VMEM((EPW,), jnp.int32),
        pltpu.VMEM((EPW,), jnp.float32),
        pltpu.VMEM((4, 80), jnp.int32),
        pltpu.VMEM((80, DH), jnp.float32),
        pltpu.SemaphoreType.DMA,
    ],
)
def _geometry(post_hbm, ei_hbm, an_hbm, embs_hbm, r2_hbm, h0s_hbm,
              px, py, pz, sidx, didx, r2b, anb, hrows, sem):
    _geom_body(post_hbm, ei_hbm, an_hbm, embs_hbm, r2_hbm, h0s_hbm,
               px, py, pz, sidx, didx, r2b, anb, hrows, sem)


def _mp_body(h_hbm, ew_hbm, srcr_hbm, dstr_hbm, out_hbm,
             idx_s, idx_d, rows0, rows1, ewv0, ewv1,
             agg_sh,
             sg0, sg1, sw0, sw1):
    cid = lax.axis_index("c")
    sid = lax.axis_index("s")
    rows = (rows0, rows1)
    ews = (ewv0, ewv1)
    sgs = (sg0, sg1)
    sws = (sw0, sw1)

    def _zero_z(i, carry):
        for k in range(4):
            rows0[i, pl.ds(k * 16, 16)] = jnp.zeros((16,), jnp.float32)
        return carry
    lax.fori_loop(0, 64, _zero_z, 0)
    for j in range(10):
        pltpu.sync_copy(rows0.at[pl.ds(0, 64)],
                        agg_sh.at[pl.ds(sid * ROWS_PER_TILE + j * 64, 64)])
    plsc.subcore_barrier()

    pltpu.sync_copy(srcr_hbm.at[cid, sid], idx_s)
    pltpu.sync_copy(dstr_hbm.at[sid], idx_d)

    def _ew_src(c):
        return ew_hbm.at[cid, pl.ds(sid * (NCH * EB) + c * EB, EB)]

    def _issue(b, c):
        pltpu.async_copy(h_hbm.at[idx_s.at[c]], rows[b], sgs[b])
        pltpu.async_copy(_ew_src(c), ews[b], sws[b])

    for b in (0, 1):
        _issue(b, b)

    def _one(b, c):
        pltpu.make_async_copy(h_hbm.at[idx_s.at[c]], rows[b], sgs[b]).wait()
        pltpu.make_async_copy(_ew_src(c), ews[b], sws[b]).wait()

        def _mul(i, carry2):
            for u in range(8):
                for k in range(4):
                    sl = pl.ds(k * 16, 16)
                    rows[b][i * 8 + u, sl] = (rows[b][i * 8 + u, sl]
                                              * ews[b][i * 8 + u, sl])
            return carry2
        lax.fori_loop(0, EB // 8, _mul, 0)
        pltpu.sync_copy(rows[b], agg_sh.at[idx_d.at[c]], add=True)

        @pl.when(c + 2 < NCH)
        def _next():
            _issue(b, c + 2)

    def _pair(g, carry):
        for b in (0, 1):
            _one(b, 2 * g + b)
        return carry
    lax.fori_loop(0, NCH // 2, _pair, 0)

    plsc.subcore_barrier()
    pltpu.sync_copy(agg_sh.at[pl.ds(sid * ROWS_PER_TILE, ROWS_PER_TILE)],
                    out_hbm.at[cid, pl.ds(sid * ROWS_PER_TILE, ROWS_PER_TILE)])


@functools.partial(
    pl.kernel,
    out_type=jax.ShapeDtypeStruct((2, N_PAD, DH), jnp.float32),
    mesh=plsc.VectorSubcoreMesh(core_axis_name="c", subcore_axis_name="s"),
    compiler_params=pltpu.CompilerParams(use_tc_tiling_on_sc=False),
    scratch_types=[
        pltpu.VMEM((NCH, EB), jnp.int32),
        pltpu.VMEM((NCH, EB), jnp.int32),
        pltpu.VMEM((EB, DH), jnp.float32),
        pltpu.VMEM((EB, DH), jnp.float32),
        pltpu.VMEM((EB, DH), jnp.float32),
        pltpu.VMEM((EB, DH), jnp.float32),
        pltpu.VMEM_SHARED((N_PAD, DH), jnp.float32),
        pltpu.SemaphoreType.DMA,
        pltpu.SemaphoreType.DMA,
        pltpu.SemaphoreType.DMA,
        pltpu.SemaphoreType.DMA,
    ],
)
def _message_pass(h_hbm, ew_hbm, srcr_hbm, dstr_hbm, out_hbm,
                  idx_s, idx_d, rows0, rows1, ewv0, ewv1,
                  agg_sh,
                  sg0, sg1, sw0, sw1):
    _mp_body(h_hbm, ew_hbm, srcr_hbm, dstr_hbm, out_hbm,
             idx_s, idx_d, rows0, rows1, ewv0, ewv1,
             agg_sh,
             sg0, sg1, sw0, sw1)


def kernel(positions, atomic_numbers, edge_index, batch, embed,
           W1_0, W2_0, Wlin_0, W1_1, W2_1, Wlin_1, Wr1, Wr2):
    src = edge_index[0]
    dst = edge_index[1]
    src_p = jnp.concatenate([src, jnp.zeros((E_PAD - N_EDGES,), src.dtype)])
    dst_p = jnp.concatenate([dst, jnp.full((E_PAD - N_EDGES,), N_NODES,
                                           dst.dtype)])
    pos_t = positions.T
    dst_g = jnp.concatenate([dst, jnp.zeros((E_PAD - N_EDGES,), dst.dtype)])
    ei = jnp.stack([src_p, dst_g]).reshape(2, 32, EPW)
    an_p = jnp.pad(atomic_numbers, (0, N_PAD - N_NODES)).reshape(32, 4, 80)
    an2 = jnp.stack([an_p, an_p + 100])
    embs = jnp.concatenate([embed[:, :DH], embed[:, DH:]], axis=0)
    srcs = src_p.reshape(16, NCH, EB)
    srcr = jnp.stack([srcs, srcs + N_PAD])
    dstr = dst_p.reshape(16, NCH, EB)
    r2, h0s = _geometry(pos_t, ei, an2, embs)
    ew0, ew1 = _edge_weights(r2, W1_0, W2_0, W1_1, W2_1)
    agg0 = _message_pass(h0s.reshape(2 * N_PAD, DH), ew0, srcr, dstr)
    h1s = _update(h0s, agg0, Wlin_0)
    agg1 = _message_pass(h1s.reshape(2 * N_PAD, DH), ew1, srcr, dstr)
    return _readout(h1s, agg1, Wlin_1, Wr1, Wr2, batch)
